# Initial kernel scaffold; baseline (speedup 1.0000x reference)
import jax
import jax.numpy as jnp
from jax.experimental import pallas as pl


def kernel(idx, table):
    raise NotImplementedError("write your pallas kernel here")


# speedup vs baseline: 1.7967x; 1.7967x over previous
import functools

import jax
import jax.numpy as jnp
from jax import lax
from jax.experimental import pallas as pl
from jax.experimental.pallas import tpu as pltpu
from jax.experimental.pallas import tpu_sc as plsc

NUM_EMB = 1000000
D = 64
B_ROWS = 16384 * 50
IW = 128
N_IDX_ROWS = B_ROWS // IW
NC, NS = 2, 16
NW = NC * NS
ROWS_PER_W = N_IDX_ROWS // NW
K = 4
NCHUNK = ROWS_PER_W // K

_mesh = plsc.VectorSubcoreMesh(
    core_axis_name="c", subcore_axis_name="s", num_cores=NC, num_subcores=NS
)


@functools.partial(
    pl.kernel,
    out_type=jax.ShapeDtypeStruct((B_ROWS, D), jnp.float32),
    mesh=_mesh,
    scratch_types=[
        pltpu.VMEM((K, IW), jnp.int32),
        pltpu.VMEM((K * IW, D), jnp.float32),
        pltpu.SemaphoreType.DMA,
    ],
    compiler_params=pltpu.CompilerParams(use_tc_tiling_on_sc=False),
)
def _gather_kernel(idx_hbm, table_hbm, out_hbm, idx_v, rows_v, sem):
    wid = lax.axis_index("s") * NC + lax.axis_index("c")
    row0 = wid * ROWS_PER_W

    @pl.loop(0, NCHUNK)
    def _chunk(g):
        rbase = row0 + g * K
        pltpu.sync_copy(idx_hbm.at[pl.ds(rbase, K)], idx_v)
        handles = [
            pltpu.async_copy(
                table_hbm.at[idx_v.at[j]], rows_v.at[pl.ds(j * IW, IW)], sem
            )
            for j in range(K)
        ]
        for h in handles:
            h.wait()
        pltpu.sync_copy(rows_v, out_hbm.at[pl.ds(rbase * IW, K * IW)])


def kernel(idx, table):
    idx2d = idx.reshape(N_IDX_ROWS, IW)
    out = _gather_kernel(idx2d, table)
    return out.reshape(idx.shape[0], idx.shape[1], D)


# speedup vs baseline: 1.8642x; 1.0375x over previous
import functools

import jax
import jax.numpy as jnp
from jax import lax
from jax.experimental import pallas as pl
from jax.experimental.pallas import tpu as pltpu
from jax.experimental.pallas import tpu_sc as plsc

NUM_EMB = 1000000
D = 64
B_ROWS = 16384 * 50
IW = 128
N_IDX_ROWS = B_ROWS // IW
NC, NS = 2, 16
NW = NC * NS
ROWS_PER_W = N_IDX_ROWS // NW
K = 4
CHUNK = K * IW
NCHUNK = ROWS_PER_W // K

_mesh = plsc.VectorSubcoreMesh(
    core_axis_name="c", subcore_axis_name="s", num_cores=NC, num_subcores=NS
)


@functools.partial(
    pl.kernel,
    out_type=jax.ShapeDtypeStruct((B_ROWS, D), jnp.float32),
    mesh=_mesh,
    scratch_types=[
        pltpu.VMEM((ROWS_PER_W, IW), jnp.int32),
        pltpu.VMEM((CHUNK, D), jnp.float32),
        pltpu.VMEM((CHUNK, D), jnp.float32),
        pltpu.SemaphoreType.DMA,
        pltpu.SemaphoreType.DMA,
        pltpu.SemaphoreType.DMA,
        pltpu.SemaphoreType.DMA,
    ],
    compiler_params=pltpu.CompilerParams(use_tc_tiling_on_sc=False),
)
def _gather_kernel(idx_hbm, table_hbm, out_hbm, idx_v, r0, r1, g0, g1, s0, s1):
    wid = lax.axis_index("s") * NC + lax.axis_index("c")
    row0 = wid * ROWS_PER_W

    pltpu.sync_copy(idx_hbm.at[pl.ds(row0, ROWS_PER_W)], idx_v)

    def fire_gather(c, buf, sem):
        for j in range(K):
            pltpu.async_copy(
                table_hbm.at[idx_v.at[c * K + j]],
                buf.at[pl.ds(j * IW, IW)],
                sem,
            )

    def wait_gather(buf, sem):
        pltpu.make_async_copy(table_hbm.at[pl.ds(0, CHUNK)], buf, sem).wait()

    def fire_store(c, buf, sem):
        pltpu.async_copy(buf, out_hbm.at[pl.ds((row0 + c * K) * IW, CHUNK)], sem)

    def wait_store(buf, sem):
        pltpu.make_async_copy(buf, out_hbm.at[pl.ds(0, CHUNK)], sem).wait()

    fire_gather(0, r0, g0)
    fire_gather(1, r1, g1)

    @pl.loop(0, NCHUNK - 2, step=2)
    def _pair(c):
        wait_gather(r0, g0)
        fire_store(c, r0, s0)
        wait_gather(r1, g1)
        fire_store(c + 1, r1, s1)
        wait_store(r0, s0)
        fire_gather(c + 2, r0, g0)
        wait_store(r1, s1)
        fire_gather(c + 3, r1, g1)

    wait_gather(r0, g0)
    fire_store(NCHUNK - 2, r0, s0)
    wait_gather(r1, g1)
    fire_store(NCHUNK - 1, r1, s1)
    wait_store(r0, s0)
    wait_store(r1, s1)


def kernel(idx, table):
    idx2d = idx.reshape(N_IDX_ROWS, IW)
    out = _gather_kernel(idx2d, table)
    return out.reshape(idx.shape[0], idx.shape[1], D)
